# baseline (speedup 1.0000x reference)
import functools

import jax
import jax.numpy as jnp
from jax import lax
from jax.experimental import pallas as pl
from jax.experimental.pallas import tpu as pltpu
from jax.experimental.pallas import tpu_sc as plsc

N = 10000
E = 320000
D = 128
NPAD = 10240

NC = 2
NS = 16
L = 16
NW = NC * NS
EW = E // NW
C = 80
NCH = EW // C
RPW = NPAD // NS
ZR = 128

_mesh = plsc.VectorSubcoreMesh(core_axis_name="c", subcore_axis_name="s")


@functools.partial(
    pl.kernel,
    out_type=jax.ShapeDtypeStruct((NW * NPAD,), jnp.float32),
    mesh=_mesh,
    compiler_params=pltpu.CompilerParams(needs_layout_passes=False),
    scratch_types=[
        pltpu.VMEM((NPAD,), jnp.float32),
        pltpu.VMEM((EW,), jnp.int32),
        pltpu.VMEM((EW,), jnp.float32),
    ],
)
def _deg(row_hbm, attr_hbm, out_hbm, acc, rows, attrs):
    c = lax.axis_index("c")
    s = lax.axis_index("s")
    gw = c * NS + s
    base = gw * EW
    pltpu.sync_copy(row_hbm.at[pl.ds(base, EW)], rows)
    pltpu.sync_copy(attr_hbm.at[pl.ds(base, EW)], attrs)

    def zero(i, carry):
        acc[pl.ds(i * L, L)] = jnp.zeros((L,), jnp.float32)
        return carry

    lax.fori_loop(0, NPAD // L, zero, 0)

    def body(i, carry):
        r = rows[pl.ds(i * L, L)]
        a = attrs[pl.ds(i * L, L)]
        plsc.addupdate_scatter(acc, [r], a)
        return carry

    lax.fori_loop(0, EW // L, body, 0)
    pltpu.sync_copy(acc, out_hbm.at[pl.ds(gw * NPAD, NPAD)])


def _dinv_body(degp_ref, dinv_ref):
    deg = jnp.sum(degp_ref[...], axis=0)
    r = lax.rsqrt(jnp.maximum(deg, 1e-12))
    dinv_ref[...] = jnp.where(deg > 0, r, 0.0)


def _dinv(degp):
    return pl.pallas_call(
        _dinv_body,
        out_shape=jax.ShapeDtypeStruct((NPAD // 128, 128), jnp.float32),
    )(degp.reshape(NW, NPAD // 128, 128))


@functools.partial(
    pl.kernel,
    out_type=jax.ShapeDtypeStruct((E,), jnp.float32),
    mesh=_mesh,
    compiler_params=pltpu.CompilerParams(needs_layout_passes=False),
    scratch_types=[
        pltpu.VMEM((NPAD,), jnp.float32),
        pltpu.VMEM((EW,), jnp.int32),
        pltpu.VMEM((EW,), jnp.int32),
        pltpu.VMEM((EW,), jnp.float32),
        pltpu.VMEM((EW,), jnp.float32),
    ],
)
def _lap(row_hbm, col_hbm, attr_hbm, dinv_hbm, lap_hbm, dinv_v, rows, cols, attrs, lap_v):
    c = lax.axis_index("c")
    s = lax.axis_index("s")
    gw = c * NS + s
    base = gw * EW
    pltpu.sync_copy(dinv_hbm, dinv_v)
    pltpu.sync_copy(row_hbm.at[pl.ds(base, EW)], rows)
    pltpu.sync_copy(col_hbm.at[pl.ds(base, EW)], cols)
    pltpu.sync_copy(attr_hbm.at[pl.ds(base, EW)], attrs)

    def body(i, carry):
        sl = pl.ds(i * L, L)
        dr = plsc.load_gather(dinv_v, [rows[sl]])
        dc = plsc.load_gather(dinv_v, [cols[sl]])
        lap_v[sl] = -(dr * attrs[sl] * dc)
        return carry

    lax.fori_loop(0, EW // L, body, 0)
    pltpu.sync_copy(lap_v, lap_hbm.at[pl.ds(base, EW)])


DH = D // NC
CS = 128
ET = E // NS
NCHW = ET // CS
CT = ET - NCHW * CS
RW0 = 624
RW1 = N - (NS - 1) * RW0


@functools.partial(
    pl.kernel,
    out_type=[jax.ShapeDtypeStruct((NC, N, DH), jnp.float32) for _ in range(3)],
    mesh=_mesh,
    compiler_params=pltpu.CompilerParams(needs_layout_passes=False),
    scratch_types=[
        pltpu.VMEM_SHARED((N, DH), jnp.float32),
        pltpu.VMEM_SHARED((N, DH), jnp.float32),
        [pltpu.VMEM((CS,), jnp.int32) for _ in range(2)],
        [pltpu.VMEM((CS,), jnp.int32) for _ in range(2)],
        [pltpu.VMEM((CS,), jnp.float32) for _ in range(2)],
        [pltpu.VMEM((CS, DH), jnp.float32) for _ in range(2)],
        pltpu.VMEM((CS // 2, DH), jnp.float32),
        pltpu.VMEM((CT,), jnp.int32),
        pltpu.VMEM((CT,), jnp.int32),
        pltpu.VMEM((CT,), jnp.float32),
        pltpu.VMEM((CT, DH), jnp.float32),
        [pltpu.SemaphoreType.DMA for _ in range(2)],
        [pltpu.SemaphoreType.DMA for _ in range(2)],
        [pltpu.SemaphoreType.DMA for _ in range(2)],
        [pltpu.SemaphoreType.DMA for _ in range(2)],
        [pltpu.SemaphoreType.DMA for _ in range(2)],
        [pltpu.SemaphoreType.DMA for _ in range(4)],
    ],
)
def _cheb(xh_hbm, col_hbm, row_hbm, lap_hbm, t1_hbm, t2_hbm, t3_hbm,
          tbl, acc, colp, rowp, lapp, rbuf, zbuf, colt, rowt, lapt, rbt,
          csem, psem, qsem, gsem, ssem, tsem):
    c = lax.axis_index("c")
    s = lax.axis_index("s")
    ebase = s * ET
    rbase = s * RW0
    nblk = 4 + (s == NS - 1)
    myrows = jnp.where(s == NS - 1, RW1, RW0)

    HB = CS // 2

    def zfill(i, carry):
        for j in range(DH // L):
            zbuf[i, pl.ds(j * L, L)] = jnp.zeros((L,), jnp.float32)
        return carry

    lax.fori_loop(0, HB, zfill, 0)

    def zero_acc(base, n):
        for q in range(n // HB):
            pltpu.sync_copy(zbuf, acc.at[pl.ds(base + q * HB, HB), :])
        rem = n % HB
        if rem:
            pltpu.sync_copy(zbuf.at[pl.ds(0, rem)],
                            acc.at[pl.ds(base + (n // HB) * HB, rem), :])

    def init_blk(i, carry):
        rows = jnp.minimum(myrows - i * CS, CS)

        @pl.when(rows == CS)
        def _():
            pltpu.sync_copy(xh_hbm.at[c, pl.ds(rbase + i * CS, CS), :],
                            tbl.at[pl.ds(rbase + i * CS, CS), :])
            zero_acc(rbase + i * CS, CS)

        @pl.when(rows < CS)
        def _():
            pltpu.sync_copy(xh_hbm.at[c, pl.ds(rbase + i * CS, RW0 - 4 * CS), :],
                            tbl.at[pl.ds(rbase + i * CS, RW0 - 4 * CS), :])
            zero_acc(rbase + i * CS, RW0 - 4 * CS)
        return carry

    lax.fori_loop(0, nblk + (s < NS - 1), init_blk, 0)
    plsc.subcore_barrier()

    def c_copy(k, b):
        return pltpu.make_async_copy(
            col_hbm.at[pl.ds(ebase + k * CS, CS)], colp[b], csem[b])

    def r_copy(k, b):
        return pltpu.make_async_copy(
            row_hbm.at[pl.ds(ebase + k * CS, CS)], rowp[b], psem[b])

    def l_copy(k, b):
        return pltpu.make_async_copy(
            lap_hbm.at[pl.ds(ebase + k * CS, CS)], lapp[b], qsem[b])

    def g_copy(b):
        return pltpu.make_async_copy(tbl.at[colp[b]], rbuf[b], gsem[b])

    def s_copy(b):
        return pltpu.make_async_copy(rbuf[b], acc.at[rowp[b]], ssem[b])

    def scale(buf, lp, n):
        def edge(i, carry):
            e = 2 * i
            lv0 = plsc.load_gather(lp, [jnp.zeros((L,), jnp.int32) + e])
            lv1 = plsc.load_gather(lp, [jnp.zeros((L,), jnp.int32) + (e + 1)])
            for j in range(DH // L):
                sl = pl.ds(j * L, L)
                buf[e, sl] = buf[e, sl] * lv0
            for j in range(DH // L):
                sl = pl.ds(j * L, L)
                buf[e + 1, sl] = buf[e + 1, sl] * lv1
            return carry

        lax.fori_loop(0, n // 2, edge, 0)

    def step(k, u):
        b = u % 2
        ob = 1 - b
        first = u < 2
        last1 = u >= NCHW - 1
        last2 = u >= NCHW - 2
        g_copy(b).wait()
        if not last2:
            c_copy(k + 2, b).start()
        r_copy(k, b).wait()
        l_copy(k, b).wait()
        if not first:
            s_copy(ob).wait()
        if not last1:
            c_copy(k + 1, ob).wait()
            g_copy(ob).start()
            r_copy(k + 1, ob).start()
            l_copy(k + 1, ob).start()
        scale(rbuf[b], lapp[b], CS)
        s_copy(b).start(add=True)

    def spmm_round(r, t_hbm, prev_hbm):
        c_copy(0, 0).start()
        c_copy(1, 1).start()
        r_copy(0, 0).start()
        l_copy(0, 0).start()
        c_copy(0, 0).wait()
        g_copy(0).start()
        step(0, 0)
        step(1, 1)

        def outer(g, carry):
            step(2 + 2 * g, 2)
            step(3 + 2 * g, 3)
            return carry

        lax.fori_loop(0, (NCHW - 4) // 2, outer, 0)
        step(NCHW - 2, NCHW - 2)
        step(NCHW - 1, NCHW - 1)
        s_copy((NCHW - 1) % 2).wait()

        tb = ebase + NCHW * CS
        pltpu.sync_copy(col_hbm.at[pl.ds(tb, CT)], colt)
        pltpu.sync_copy(row_hbm.at[pl.ds(tb, CT)], rowt)
        pltpu.sync_copy(lap_hbm.at[pl.ds(tb, CT)], lapt)
        pltpu.async_copy(tbl.at[colt], rbt, tsem[0]).wait()
        scale(rbt, lapt, CT)
        pltpu.async_copy(rbt, acc.at[rowt], tsem[1], add=True).wait()
        plsc.subcore_barrier()

        def comb_blk(i, carry):
            rows = jnp.minimum(myrows - i * CS, CS)
            npart = RW0 - 4 * CS

            @pl.when(rows == CS)
            def _():
                base = rbase + i * CS
                pltpu.sync_copy(acc.at[pl.ds(base, CS), :], rbuf[0])
                if r > 1:
                    pltpu.sync_copy(prev_hbm.at[c, pl.ds(base, CS), :], rbuf[1])

                    def fix(q, carry2):
                        for j in range(DH // L):
                            sl = pl.ds(j * L, L)
                            rbuf[0][q, sl] = rbuf[0][q, sl] * 2.0 - rbuf[1][q, sl]
                        return carry2

                    lax.fori_loop(0, CS, fix, 0)
                pltpu.sync_copy(rbuf[0], t_hbm.at[c, pl.ds(base, CS), :])
                if r < 3:
                    pltpu.sync_copy(rbuf[0], tbl.at[pl.ds(base, CS), :])
                    zero_acc(base, CS)

            @pl.when(rows < CS)
            def _():
                base = rbase + i * CS
                pltpu.sync_copy(acc.at[pl.ds(base, npart), :],
                                rbuf[0].at[pl.ds(0, npart)])
                if r > 1:
                    pltpu.sync_copy(prev_hbm.at[c, pl.ds(base, npart), :],
                                    rbuf[1].at[pl.ds(0, npart)])

                    def fix(q, carry2):
                        for j in range(DH // L):
                            sl = pl.ds(j * L, L)
                            rbuf[0][q, sl] = rbuf[0][q, sl] * 2.0 - rbuf[1][q, sl]
                        return carry2

                    lax.fori_loop(0, npart, fix, 0)
                pltpu.sync_copy(rbuf[0].at[pl.ds(0, npart)],
                                t_hbm.at[c, pl.ds(base, npart), :])
                if r < 3:
                    pltpu.sync_copy(rbuf[0].at[pl.ds(0, npart)],
                                    tbl.at[pl.ds(base, npart), :])
                    zero_acc(base, npart)
            return carry

        lax.fori_loop(0, nblk + (s < NS - 1), comb_blk, 0)
        plsc.subcore_barrier()

    spmm_round(1, t1_hbm, None)
    spmm_round(2, t2_hbm, xh_hbm)
    spmm_round(3, t3_hbm, t1_hbm)


_RB = 400


def _final_body(x_ref, t1_ref, t2_ref, t3_ref, w_ref, b_ref, out_ref):
    t1 = jnp.concatenate([t1_ref[0], t1_ref[1]], axis=1)
    t2 = jnp.concatenate([t2_ref[0], t2_ref[1]], axis=1)
    t3 = jnp.concatenate([t3_ref[0], t3_ref[1]], axis=1)
    w = w_ref[...]
    acc = jnp.dot(x_ref[...], w[0], preferred_element_type=jnp.float32)
    acc += jnp.dot(t1, w[1], preferred_element_type=jnp.float32)
    acc += jnp.dot(t2, w[2], preferred_element_type=jnp.float32)
    acc += jnp.dot(t3, w[3], preferred_element_type=jnp.float32)
    out_ref[...] = acc + b_ref[...]


def _final(x, t1h, t2h, t3h, weight, bias):
    grid = N // _RB
    bs = pl.BlockSpec((_RB, D), lambda i: (i, 0))
    hs = pl.BlockSpec((NC, _RB, DH), lambda i: (0, i, 0))
    return pl.pallas_call(
        _final_body,
        grid=(grid,),
        in_specs=[
            bs, hs, hs, hs,
            pl.BlockSpec((4, D, D), lambda i: (0, 0, 0)),
            pl.BlockSpec((1, D), lambda i: (0, 0)),
        ],
        out_specs=bs,
        out_shape=jax.ShapeDtypeStruct((N, D), jnp.float32),
    )(x, t1h, t2h, t3h, weight, bias.reshape(1, D))


def kernel(x, edge_index, edge_attr, weight, bias):
    row = edge_index[0]
    col = edge_index[1]
    degp = _deg(row, edge_attr)
    dinv = _dinv(degp.reshape(NW, NPAD)).reshape(NPAD)
    lap = _lap(row, col, edge_attr, dinv)
    xh = jnp.stack([x[:, :DH], x[:, DH:]])
    t1h, t2h, t3h = _cheb(xh, col, row, lap)
    return _final(x, t1h, t2h, t3h, weight, bias)


# speedup vs baseline: 1.0944x; 1.0944x over previous
import functools

import jax
import jax.numpy as jnp
from jax import lax
from jax.experimental import pallas as pl
from jax.experimental.pallas import tpu as pltpu
from jax.experimental.pallas import tpu_sc as plsc

N = 10000
E = 320000
D = 128
NPAD = 10240

NC = 2
NS = 16
L = 16
NW = NC * NS
EW = E // NW
C = 80
NCH = EW // C
RPW = NPAD // NS
ZR = 128

_mesh = plsc.VectorSubcoreMesh(core_axis_name="c", subcore_axis_name="s")


@functools.partial(
    pl.kernel,
    out_type=jax.ShapeDtypeStruct((NW * NPAD,), jnp.float32),
    mesh=_mesh,
    compiler_params=pltpu.CompilerParams(needs_layout_passes=False),
    scratch_types=[
        pltpu.VMEM((NPAD,), jnp.float32),
        pltpu.VMEM((EW,), jnp.int32),
        pltpu.VMEM((EW,), jnp.float32),
    ],
)
def _deg(row_hbm, attr_hbm, out_hbm, acc, rows, attrs):
    c = lax.axis_index("c")
    s = lax.axis_index("s")
    gw = c * NS + s
    base = gw * EW
    pltpu.sync_copy(row_hbm.at[pl.ds(base, EW)], rows)
    pltpu.sync_copy(attr_hbm.at[pl.ds(base, EW)], attrs)

    def zero(i, carry):
        acc[pl.ds(i * L, L)] = jnp.zeros((L,), jnp.float32)
        return carry

    lax.fori_loop(0, NPAD // L, zero, 0)

    def body(i, carry):
        r = rows[pl.ds(i * L, L)]
        a = attrs[pl.ds(i * L, L)]
        plsc.addupdate_scatter(acc, [r], a)
        return carry

    lax.fori_loop(0, EW // L, body, 0)
    pltpu.sync_copy(acc, out_hbm.at[pl.ds(gw * NPAD, NPAD)])


def _dinv_body(degp_ref, dinv_ref):
    deg = jnp.sum(degp_ref[...], axis=0)
    r = lax.rsqrt(jnp.maximum(deg, 1e-12))
    dinv_ref[...] = jnp.where(deg > 0, r, 0.0)


def _dinv(degp):
    return pl.pallas_call(
        _dinv_body,
        out_shape=jax.ShapeDtypeStruct((NPAD // 128, 128), jnp.float32),
    )(degp.reshape(NW, NPAD // 128, 128))


@functools.partial(
    pl.kernel,
    out_type=jax.ShapeDtypeStruct((E,), jnp.float32),
    mesh=_mesh,
    compiler_params=pltpu.CompilerParams(needs_layout_passes=False),
    scratch_types=[
        pltpu.VMEM((NPAD,), jnp.float32),
        pltpu.VMEM((EW,), jnp.int32),
        pltpu.VMEM((EW,), jnp.int32),
        pltpu.VMEM((EW,), jnp.float32),
        pltpu.VMEM((EW,), jnp.float32),
    ],
)
def _lap(row_hbm, col_hbm, attr_hbm, dinv_hbm, lap_hbm, dinv_v, rows, cols, attrs, lap_v):
    c = lax.axis_index("c")
    s = lax.axis_index("s")
    gw = c * NS + s
    base = gw * EW
    pltpu.sync_copy(dinv_hbm, dinv_v)
    pltpu.sync_copy(row_hbm.at[pl.ds(base, EW)], rows)
    pltpu.sync_copy(col_hbm.at[pl.ds(base, EW)], cols)
    pltpu.sync_copy(attr_hbm.at[pl.ds(base, EW)], attrs)

    def body(i, carry):
        sl = pl.ds(i * L, L)
        dr = plsc.load_gather(dinv_v, [rows[sl]])
        dc = plsc.load_gather(dinv_v, [cols[sl]])
        lap_v[sl] = -(dr * attrs[sl] * dc)
        return carry

    lax.fori_loop(0, EW // L, body, 0)
    pltpu.sync_copy(lap_v, lap_hbm.at[pl.ds(base, EW)])


CS = 128
NCHW = 78
TAIL = E - NW * NCHW * CS
RW0 = 624
RW1 = N - (NS - 1) * RW0


@functools.partial(
    pl.kernel,
    out_type=jax.ShapeDtypeStruct((NC, N, D), jnp.float32),
    mesh=_mesh,
    compiler_params=pltpu.CompilerParams(needs_layout_passes=False),
    scratch_types=[
        pltpu.VMEM_SHARED((N, D), jnp.float32),
        [pltpu.VMEM((CS,), jnp.int32) for _ in range(3)],
        [pltpu.VMEM((CS,), jnp.int32) for _ in range(3)],
        [pltpu.VMEM((CS,), jnp.float32) for _ in range(3)],
        [pltpu.VMEM((CS, D), jnp.float32) for _ in range(3)],
        [pltpu.SemaphoreType.DMA for _ in range(3)],
        [pltpu.SemaphoreType.DMA for _ in range(3)],
        [pltpu.SemaphoreType.DMA for _ in range(3)],
        [pltpu.SemaphoreType.DMA for _ in range(3)],
        [pltpu.SemaphoreType.DMA for _ in range(3)],
        [pltpu.SemaphoreType.DMA for _ in range(2)],
    ],
)
def _spmm(m_hbm, col_hbm, row_hbm, lap_hbm, out_hbm, acc,
          colp, rowp, lapp, rbuf, csem, psem, qsem, gsem, ssem, tsem):
    c = lax.axis_index("c")
    s = lax.axis_index("s")
    gw = c * NS + s
    ebase = gw * NCHW * CS

    def zfill(i, carry):
        for j in range(D // L):
            rbuf[0][i, pl.ds(j * L, L)] = jnp.zeros((L,), jnp.float32)
        return carry

    lax.fori_loop(0, CS, zfill, 0)

    @pl.when(s < NS - 1)
    def _():
        for k in range(4):
            pltpu.sync_copy(rbuf[0], acc.at[pl.ds(s * RW0 + k * CS, CS), :])
        pltpu.sync_copy(rbuf[0].at[pl.ds(0, RW0 - 4 * CS)],
                        acc.at[pl.ds(s * RW0 + 4 * CS, RW0 - 4 * CS), :])

    @pl.when(s == NS - 1)
    def _():
        for k in range(RW1 // CS):
            pltpu.sync_copy(rbuf[0], acc.at[pl.ds(s * RW0 + k * CS, CS), :])

    plsc.subcore_barrier()

    def c_copy(k, b):
        return pltpu.make_async_copy(
            col_hbm.at[pl.ds(ebase + k * CS, CS)], colp[b], csem[b])

    def r_copy(k, b):
        return pltpu.make_async_copy(
            row_hbm.at[pl.ds(ebase + k * CS, CS)], rowp[b], psem[b])

    def l_copy(k, b):
        return pltpu.make_async_copy(
            lap_hbm.at[pl.ds(ebase + k * CS, CS)], lapp[b], qsem[b])

    def g_copy(b2, b3):
        return pltpu.make_async_copy(m_hbm.at[colp[b2]], rbuf[b3], gsem[b3])

    def s_copy(b3):
        return pltpu.make_async_copy(rbuf[b3], acc.at[rowp[b3]], ssem[b3])

    def scale(b3, b2):
        buf = rbuf[b3]
        lp = lapp[b2]

        def edge(i, carry):
            e = 2 * i
            lv0 = plsc.load_gather(lp, [jnp.zeros((L,), jnp.int32) + e])
            lv1 = plsc.load_gather(lp, [jnp.zeros((L,), jnp.int32) + (e + 1)])
            for j in range(D // L):
                sl = pl.ds(j * L, L)
                buf[e, sl] = buf[e, sl] * lv0
            for j in range(D // L):
                sl = pl.ds(j * L, L)
                buf[e + 1, sl] = buf[e + 1, sl] * lv1
            return carry

        lax.fori_loop(0, CS // 2, edge, 0)

    def step(k, u):
        b3 = u % 3
        first = u < 1
        g_copy(b3, b3).wait()
        if u + 3 < NCHW:
            c_copy(k + 3, b3).start()
        r_copy(k, b3).wait()
        l_copy(k, b3).wait()
        if not first:
            s_copy((u + 2) % 3).wait()
        if u + 2 < NCHW:
            nb = (u + 2) % 3
            c_copy(k + 2, nb).wait()
            g_copy(nb, nb).start()
            r_copy(k + 2, nb).start()
            l_copy(k + 2, nb).start()
        scale(b3, b3)
        s_copy(b3).start(add=True)

    c_copy(0, 0).start()
    c_copy(1, 1).start()
    c_copy(2, 2).start()
    r_copy(0, 0).start()
    l_copy(0, 0).start()
    r_copy(1, 1).start()
    l_copy(1, 1).start()
    c_copy(0, 0).wait()
    g_copy(0, 0).start()
    c_copy(1, 1).wait()
    g_copy(1, 1).start()
    step(0, 0)
    step(1, 1)

    def outer(g, carry):
        for u in range(3):
            step(2 + 3 * g + u, 2 + u)
        return carry

    lax.fori_loop(0, 24, outer, 0)
    for k in range(74, NCHW):
        step(k, k)
    s_copy((NCHW - 1) % 3).wait()

    @pl.when(gw * CS < TAIL)
    def _():
        tb = (NW * NCHW + gw) * CS
        pltpu.sync_copy(col_hbm.at[pl.ds(tb, CS)], colp[0])
        pltpu.sync_copy(row_hbm.at[pl.ds(tb, CS)], rowp[0])
        pltpu.sync_copy(lap_hbm.at[pl.ds(tb, CS)], lapp[0])
        pltpu.async_copy(m_hbm.at[colp[0]], rbuf[0], tsem[0]).wait()
        scale(0, 0)
        pltpu.async_copy(rbuf[0], acc.at[rowp[0]], tsem[1], add=True).wait()

    plsc.subcore_barrier()

    @pl.when(s < NS - 1)
    def _():
        pltpu.sync_copy(acc.at[pl.ds(s * RW0, RW0), :],
                        out_hbm.at[c, pl.ds(s * RW0, RW0), :])

    @pl.when(s == NS - 1)
    def _():
        pltpu.sync_copy(acc.at[pl.ds(s * RW0, RW1), :],
                        out_hbm.at[c, pl.ds(s * RW0, RW1), :])


_RB = 400


def _combine_body(a, b, p0_ref, p1_ref, prev_ref, out_ref):
    out_ref[...] = a * (p0_ref[...] + p1_ref[...]) + b * prev_ref[...]


def _combine(p0, p1, prev, a, b):
    grid = N // _RB
    bs = pl.BlockSpec((_RB, D), lambda i: (i, 0))
    return pl.pallas_call(
        functools.partial(_combine_body, a, b),
        grid=(grid,),
        in_specs=[bs, bs, bs],
        out_specs=bs,
        out_shape=jax.ShapeDtypeStruct((N, D), jnp.float32),
    )(p0, p1, prev)


def _final_body(x_ref, t1_ref, t2_ref, p0_ref, p1_ref, w_ref, b_ref, out_ref):
    t3 = 2.0 * (p0_ref[...] + p1_ref[...]) - t1_ref[...]
    w = w_ref[...]
    acc = jnp.dot(x_ref[...], w[0], preferred_element_type=jnp.float32)
    acc += jnp.dot(t1_ref[...], w[1], preferred_element_type=jnp.float32)
    acc += jnp.dot(t2_ref[...], w[2], preferred_element_type=jnp.float32)
    acc += jnp.dot(t3, w[3], preferred_element_type=jnp.float32)
    out_ref[...] = acc + b_ref[...]


def _final(x, t1, t2, p0, p1, weight, bias):
    grid = N // _RB
    bs = pl.BlockSpec((_RB, D), lambda i: (i, 0))
    return pl.pallas_call(
        _final_body,
        grid=(grid,),
        in_specs=[
            bs, bs, bs, bs, bs,
            pl.BlockSpec((4, D, D), lambda i: (0, 0, 0)),
            pl.BlockSpec((1, D), lambda i: (0, 0)),
        ],
        out_specs=bs,
        out_shape=jax.ShapeDtypeStruct((N, D), jnp.float32),
    )(x, t1, t2, p0, p1, weight, bias.reshape(1, D))


def kernel(x, edge_index, edge_attr, weight, bias):
    row = edge_index[0]
    col = edge_index[1]
    degp = _deg(row, edge_attr)
    dinv = _dinv(degp.reshape(NW, NPAD)).reshape(NPAD)
    lap = _lap(row, col, edge_attr, dinv)
    p = _spmm(x, col, row, lap)
    t1 = _combine(p[0], p[1], x, 1.0, 0.0)
    p = _spmm(t1, col, row, lap)
    t2 = _combine(p[0], p[1], x, 2.0, -1.0)
    p = _spmm(t2, col, row, lap)
    return _final(x, t1, t2, p[0], p[1], weight, bias)
